# baseline (speedup 1.0000x reference)
import jax
import jax.numpy as jnp
from jax import lax
from jax.experimental import pallas as pl
from jax.experimental.pallas import tpu as pltpu
from jax.experimental.pallas import tpu_sc as plsc

_B, _N, _E, _D, _R = 32, 512, 8192, 256, 8
_NG, _NS = 10000, 2000
_L = 16


def _sc_body(x2_hbm, ei_hbm, et_hbm, s_hbm, deg0_hbm,
             src_v, dst_v, et_v, histd_v, srcl_v, etl_v, s_v, rows_v, d0_v,
             sem):
    c = lax.axis_index("c")
    sx = lax.axis_index("s")
    w = sx * 2 + c

    pltpu.sync_copy(ei_hbm.at[w, 0], src_v)
    pltpu.sync_copy(ei_hbm.at[w, 1], dst_v)
    pltpu.sync_copy(et_hbm.at[w], et_v)

    zeros = jnp.zeros((_L,), jnp.float32)

    def zero_hist(i, cr):
        histd_v[pl.ds(i * _L, _L)] = zeros
        return cr

    lax.fori_loop(0, (_R * _N) // _L, zero_hist, 0, unroll=4)

    def zero_s(i, cr):
        s_v[pl.ds(i * _L, _L)] = zeros
        return cr

    lax.fori_loop(0, (_R * _D) // _L, zero_s, 0, unroll=4)

    ones = jnp.ones((_L,), jnp.float32)
    lanes = lax.iota(jnp.int32, _L)
    cap = _E // _L
    basevec = lanes * cap

    def edge_body(i, cntl):
        off = i * _L
        sv = src_v[pl.ds(off, _L)]
        dv = dst_v[pl.ds(off, _L)]
        tv = et_v[pl.ds(off, _L)]
        plsc.addupdate_scatter(histd_v, [tv * _N + dv], ones)
        m = dv == 0
        pos = basevec + cntl
        plsc.store_scatter(srcl_v, [pos], sv, mask=m)
        plsc.store_scatter(etl_v, [pos], tv, mask=m)
        return cntl + m.astype(jnp.int32)

    cntl = lax.fori_loop(0, _E // _L, edge_body,
                         jnp.zeros((_L,), jnp.int32), unroll=2)

    def chunk_body(k, cr):
        act = k < cntl
        idx = basevec + k
        sl = plsc.load_gather(srcl_v, [idx])
        tl = plsc.load_gather(etl_v, [idx])
        sl = jnp.where(act, sl, 0)
        tl = jnp.where(act, tl, 0)
        d = plsc.load_gather(histd_v, [tl * _N + sl]) + 1.0
        di = plsc.bitcast(d, jnp.int32)
        y = plsc.bitcast(jnp.int32(0x5F3759DF) - (di >> 1), jnp.float32)
        half = 0.5 * d
        y = y * (1.5 - half * y * y)
        y = y * (1.5 - half * y * y)
        y = y * (1.5 - half * y * y)
        wgt = jnp.where(act, y, jnp.float32(0.0))
        gidx = jnp.where(act, w * _N + sl, 0)
        pltpu.async_copy(x2_hbm.at[gidx], rows_v, sem).wait()

        def col_body(cc, cr2):
            colv = plsc.load_gather(rows_v, [lanes, jnp.broadcast_to(cc, (_L,))])
            plsc.addupdate_scatter(s_v, [tl * _D + cc], wgt * colv, mask=act)
            return cr2

        lax.fori_loop(0, _D, col_body, 0, unroll=4)
        return cr

    lax.fori_loop(0, jnp.max(cntl), chunk_body, 0)

    d0_v[...] = plsc.load_gather(histd_v, [jnp.minimum(lanes * _N, _R * _N - 1)])
    pltpu.sync_copy(d0_v, deg0_hbm.at[w])
    pltpu.sync_copy(s_v, s_hbm.at[w])


def _sc_gather_sums(x, edge_index, edge_type):
    x2 = x.reshape(_B * _N, _D)
    mesh = plsc.VectorSubcoreMesh(core_axis_name="c", subcore_axis_name="s")
    fn = pl.kernel(
        _sc_body,
        out_type=[
            jax.ShapeDtypeStruct((_B, _R * _D), jnp.float32),
            jax.ShapeDtypeStruct((_B, _L), jnp.float32),
        ],
        mesh=mesh,
        compiler_params=pltpu.CompilerParams(needs_layout_passes=False),
        scratch_types=[
            pltpu.VMEM((_E,), jnp.int32),
            pltpu.VMEM((_E,), jnp.int32),
            pltpu.VMEM((_E,), jnp.int32),
            pltpu.VMEM((_R * _N,), jnp.float32),
            pltpu.VMEM((_E,), jnp.int32),
            pltpu.VMEM((_E,), jnp.int32),
            pltpu.VMEM((_R * _D,), jnp.float32),
            pltpu.VMEM((_L, _D), jnp.float32),
            pltpu.VMEM((_L,), jnp.float32),
            pltpu.SemaphoreType.DMA,
        ],
    )
    return fn(x2, edge_index, edge_type)


def _tc_body(s_ref, d0_ref, x0_ref, wrel_ref, w0_ref,
             wg_ref, bg_ref, ws_ref, bs_ref, lg_ref, ls_ref):
    dinv0 = lax.rsqrt(d0_ref[...] + 1.0)
    x0 = x0_ref[...]
    acc = jnp.dot(x0, w0_ref[...], preferred_element_type=jnp.float32)
    for r in range(_R):
        dr = dinv0[:, r:r + 1]
        p = s_ref[:, r, :] * dr + x0 * (dr * dr)
        acc = acc + jnp.dot(p, wrel_ref[r], preferred_element_type=jnp.float32)
    x1 = jnp.where(acc >= 0, acc, 0.1 * acc)

    def head(w_ref, b_ref):
        z = jnp.dot(x1, w_ref[...], preferred_element_type=jnp.float32) + b_ref[...]
        m = jnp.max(z, axis=1, keepdims=True)
        e = jnp.exp(z - m)
        return z - m - jnp.log(jnp.sum(e, axis=1, keepdims=True))

    lg_ref[...] = head(wg_ref, bg_ref)
    ls_ref[...] = head(ws_ref, bs_ref)


def _tc_call(s, deg0, x0, W_rel, W_0, W_g, b_g, W_s, b_s):
    out_shape = [
        jax.ShapeDtypeStruct((_B, _NG), jnp.float32),
        jax.ShapeDtypeStruct((_B, _NS), jnp.float32),
    ]
    return pl.pallas_call(_tc_body, out_shape=out_shape)(
        s, deg0, x0, W_rel, W_0, W_g,
        b_g.reshape(1, _NG), W_s, b_s.reshape(1, _NS))


def kernel(x, edge_index, edge_type, W_rel, W_0, W_g, b_g, W_s, b_s):
    s_flat, deg0 = _sc_gather_sums(x, edge_index, edge_type)
    s = s_flat.reshape(_B, _R, _D)
    lg, ls = _tc_call(s, deg0, x[:, 0, :], W_rel, W_0, W_g, b_g, W_s, b_s)
    return lg, ls


# speedup vs baseline: 1.4380x; 1.4380x over previous
import jax
import jax.numpy as jnp
from jax import lax
from jax.experimental import pallas as pl
from jax.experimental.pallas import tpu as pltpu
from jax.experimental.pallas import tpu_sc as plsc

_B, _N, _E, _D, _R = 32, 512, 8192, 256, 8
_NG, _NS = 10000, 2000
_L = 16


def _sc_body(x2_hbm, ei_hbm, et_hbm, s_hbm, deg0_hbm,
             src_v, dst_v, et_v, histd_v, srcl_v, etl_v, s_v, rows_v, d0_v,
             sem):
    c = lax.axis_index("c")
    sx = lax.axis_index("s")
    w = sx * 2 + c

    pltpu.sync_copy(ei_hbm.at[w, 0], src_v)
    pltpu.sync_copy(ei_hbm.at[w, 1], dst_v)
    pltpu.sync_copy(et_hbm.at[w], et_v)

    zeros = jnp.zeros((_L,), jnp.float32)

    def zero_hist(i, cr):
        histd_v[pl.ds(i * _L, _L)] = zeros
        return cr

    lax.fori_loop(0, (_R * _N) // _L, zero_hist, 0)

    def zero_s(i, cr):
        s_v[pl.ds(i * _L, _L)] = zeros
        return cr

    lax.fori_loop(0, (_R * _D) // _L, zero_s, 0)

    ones = jnp.ones((_L,), jnp.float32)
    lanes = lax.iota(jnp.int32, _L)

    def edge_body(i, cnt):
        off = i * _L
        sv = src_v[pl.ds(off, _L)]
        dv = dst_v[pl.ds(off, _L)]
        tv = et_v[pl.ds(off, _L)]
        plsc.addupdate_scatter(histd_v, [tv * _N + dv], ones)
        m = dv == 0
        pos = cnt + plsc.cumsum(m.astype(jnp.int32)) - 1
        plsc.store_scatter(srcl_v, [pos], sv, mask=m)
        plsc.store_scatter(etl_v, [pos], tv, mask=m)
        return cnt + plsc.all_reduce_population_count(m)

    cntv = lax.fori_loop(0, _E // _L, edge_body, jnp.zeros((_L,), jnp.int32))
    nlist = jnp.max(cntv)

    def chunk_body(k, cr):
        off = k * _L
        sl = srcl_v[pl.ds(off, _L)]
        tl = etl_v[pl.ds(off, _L)]
        act = (off + lanes) < nlist
        sl = jnp.where(act, sl, 0)
        tl = jnp.where(act, tl, 0)
        d = plsc.load_gather(histd_v, [tl * _N + sl]) + 1.0
        di = plsc.bitcast(d, jnp.int32)
        y = plsc.bitcast(jnp.int32(0x5F3759DF) - (di >> 1), jnp.float32)
        half = 0.5 * d
        y = y * (1.5 - half * y * y)
        y = y * (1.5 - half * y * y)
        y = y * (1.5 - half * y * y)
        wgt = jnp.where(act, y, jnp.float32(0.0))
        gidx = jnp.where(act, w * _N + sl, 0)
        pltpu.async_copy(x2_hbm.at[gidx], rows_v, sem).wait()

        def col_body(cc, cr2):
            colv = plsc.load_gather(rows_v, [lanes, jnp.broadcast_to(cc, (_L,))])
            plsc.addupdate_scatter(s_v, [tl * _D + cc], wgt * colv, mask=act)
            return cr2

        lax.fori_loop(0, _D, col_body, 0)
        return cr

    lax.fori_loop(0, (nlist + _L - 1) // _L, chunk_body, 0)

    d0_v[...] = plsc.load_gather(histd_v, [jnp.minimum(lanes * _N, _R * _N - 1)])
    pltpu.sync_copy(d0_v, deg0_hbm.at[w])
    pltpu.sync_copy(s_v, s_hbm.at[w])


def _sc_gather_sums(x, edge_index, edge_type):
    x2 = x.reshape(_B * _N, _D)
    mesh = plsc.VectorSubcoreMesh(core_axis_name="c", subcore_axis_name="s")
    fn = pl.kernel(
        _sc_body,
        out_type=[
            jax.ShapeDtypeStruct((_B, _R * _D), jnp.float32),
            jax.ShapeDtypeStruct((_B, _L), jnp.float32),
        ],
        mesh=mesh,
        compiler_params=pltpu.CompilerParams(needs_layout_passes=False),
        scratch_types=[
            pltpu.VMEM((_E,), jnp.int32),
            pltpu.VMEM((_E,), jnp.int32),
            pltpu.VMEM((_E,), jnp.int32),
            pltpu.VMEM((_R * _N,), jnp.float32),
            pltpu.VMEM((_E,), jnp.int32),
            pltpu.VMEM((_E,), jnp.int32),
            pltpu.VMEM((_R * _D,), jnp.float32),
            pltpu.VMEM((_L, _D), jnp.float32),
            pltpu.VMEM((_L,), jnp.float32),
            pltpu.SemaphoreType.DMA,
        ],
    )
    return fn(x2, edge_index, edge_type)


def _tc_body(s_ref, d0_ref, x0_ref, wrel_ref, w0_ref,
             wg_ref, bg_ref, ws_ref, bs_ref, lg_ref, ls_ref):
    dinv0 = lax.rsqrt(d0_ref[...] + 1.0)
    x0 = x0_ref[...]
    acc = jnp.dot(x0, w0_ref[...], preferred_element_type=jnp.float32)
    for r in range(_R):
        dr = dinv0[:, r:r + 1]
        p = s_ref[:, r, :] * dr + x0 * (dr * dr)
        acc = acc + jnp.dot(p, wrel_ref[r], preferred_element_type=jnp.float32)
    x1 = jnp.where(acc >= 0, acc, 0.1 * acc)

    def head(w_ref, b_ref):
        z = jnp.dot(x1, w_ref[...], preferred_element_type=jnp.float32) + b_ref[...]
        m = jnp.max(z, axis=1, keepdims=True)
        e = jnp.exp(z - m)
        return z - m - jnp.log(jnp.sum(e, axis=1, keepdims=True))

    lg_ref[...] = head(wg_ref, bg_ref)
    ls_ref[...] = head(ws_ref, bs_ref)


def _tc_call(s, deg0, x0, W_rel, W_0, W_g, b_g, W_s, b_s):
    out_shape = [
        jax.ShapeDtypeStruct((_B, _NG), jnp.float32),
        jax.ShapeDtypeStruct((_B, _NS), jnp.float32),
    ]
    return pl.pallas_call(_tc_body, out_shape=out_shape)(
        s, deg0, x0, W_rel, W_0, W_g,
        b_g.reshape(1, _NG), W_s, b_s.reshape(1, _NS))


def kernel(x, edge_index, edge_type, W_rel, W_0, W_g, b_g, W_s, b_s):
    s_flat, deg0 = _sc_gather_sums(x, edge_index, edge_type)
    s = s_flat.reshape(_B, _R, _D)
    lg, ls = _tc_call(s, deg0, x[:, 0, :], W_rel, W_0, W_g, b_g, W_s, b_s)
    return lg, ls


# speedup vs baseline: 1.6529x; 1.1494x over previous
import jax
import jax.numpy as jnp
from jax import lax
from jax.experimental import pallas as pl
from jax.experimental.pallas import tpu as pltpu
from jax.experimental.pallas import tpu_sc as plsc

_B, _N, _E, _D, _R = 32, 512, 8192, 256, 8
_NG, _NS = 10000, 2000
_L = 16


def _sc_body(x2_hbm, ei_hbm, et_hbm, s_hbm, deg0_hbm,
             src_v, dst_v, et_v, histd_v, srcl_v, etl_v, s_v, rows_v, d0_v,
             sem):
    c = lax.axis_index("c")
    sx = lax.axis_index("s")
    w = sx * 2 + c

    pltpu.sync_copy(ei_hbm.at[w, 0], src_v)
    pltpu.sync_copy(ei_hbm.at[w, 1], dst_v)
    pltpu.sync_copy(et_hbm.at[w], et_v)

    zeros = jnp.zeros((_L,), jnp.float32)

    @plsc.parallel_loop(0, (_R * _N) // _L, 1, unroll=4)
    def _(i):
        histd_v[pl.ds(i * _L, _L)] = zeros

    @plsc.parallel_loop(0, (_R * _D) // _L, 1, unroll=4)
    def _(i):
        s_v[pl.ds(i * _L, _L)] = zeros

    ones = jnp.ones((_L,), jnp.float32)
    lanes = lax.iota(jnp.int32, _L)

    @plsc.parallel_loop(0, _E // _L, 1, unroll=2,
                        carry=jnp.zeros((_L,), jnp.int32))
    def cntv(i, cnt):
        off = i * _L
        sv = src_v[pl.ds(off, _L)]
        dv = dst_v[pl.ds(off, _L)]
        tv = et_v[pl.ds(off, _L)]
        plsc.addupdate_scatter(histd_v, [tv * _N + dv], ones)
        m = dv == 0
        pos = cnt + plsc.cumsum(m.astype(jnp.int32)) - 1
        plsc.store_scatter(srcl_v, [pos], sv, mask=m)
        plsc.store_scatter(etl_v, [pos], tv, mask=m)
        return cnt + plsc.all_reduce_population_count(m)

    nlist = jnp.max(cntv)

    def chunk_body(k, cr):
        off = k * _L
        sl = srcl_v[pl.ds(off, _L)]
        tl = etl_v[pl.ds(off, _L)]
        act = (off + lanes) < nlist
        sl = jnp.where(act, sl, 0)
        tl = jnp.where(act, tl, 0)
        d = plsc.load_gather(histd_v, [tl * _N + sl]) + 1.0
        di = plsc.bitcast(d, jnp.int32)
        y = plsc.bitcast(jnp.int32(0x5F3759DF) - (di >> 1), jnp.float32)
        half = 0.5 * d
        y = y * (1.5 - half * y * y)
        y = y * (1.5 - half * y * y)
        y = y * (1.5 - half * y * y)
        wgt = jnp.where(act, y, jnp.float32(0.0))
        gidx = jnp.where(act, w * _N + sl, 0)
        pltpu.async_copy(x2_hbm.at[gidx], rows_v, sem).wait()

        @plsc.parallel_loop(0, _D, 1, unroll=4)
        def _(cc):
            colv = plsc.load_gather(rows_v, [lanes, jnp.broadcast_to(cc, (_L,))])
            plsc.addupdate_scatter(s_v, [tl * _D + cc], wgt * colv, mask=act)

        return cr

    lax.fori_loop(0, (nlist + _L - 1) // _L, chunk_body, 0)

    d0_v[...] = plsc.load_gather(histd_v, [jnp.minimum(lanes * _N, _R * _N - 1)])
    pltpu.sync_copy(d0_v, deg0_hbm.at[w])
    pltpu.sync_copy(s_v, s_hbm.at[w])


def _sc_gather_sums(x, edge_index, edge_type):
    x2 = x.reshape(_B * _N, _D)
    mesh = plsc.VectorSubcoreMesh(core_axis_name="c", subcore_axis_name="s")
    fn = pl.kernel(
        _sc_body,
        out_type=[
            jax.ShapeDtypeStruct((_B, _R * _D), jnp.float32),
            jax.ShapeDtypeStruct((_B, _L), jnp.float32),
        ],
        mesh=mesh,
        compiler_params=pltpu.CompilerParams(needs_layout_passes=False),
        scratch_types=[
            pltpu.VMEM((_E,), jnp.int32),
            pltpu.VMEM((_E,), jnp.int32),
            pltpu.VMEM((_E,), jnp.int32),
            pltpu.VMEM((_R * _N,), jnp.float32),
            pltpu.VMEM((_E,), jnp.int32),
            pltpu.VMEM((_E,), jnp.int32),
            pltpu.VMEM((_R * _D,), jnp.float32),
            pltpu.VMEM((_L, _D), jnp.float32),
            pltpu.VMEM((_L,), jnp.float32),
            pltpu.SemaphoreType.DMA,
        ],
    )
    return fn(x2, edge_index, edge_type)


def _tc_body(s_ref, d0_ref, x0_ref, wrel_ref, w0_ref,
             wg_ref, bg_ref, ws_ref, bs_ref, lg_ref, ls_ref):
    dinv0 = lax.rsqrt(d0_ref[...] + 1.0)
    x0 = x0_ref[...]
    acc = jnp.dot(x0, w0_ref[...], preferred_element_type=jnp.float32)
    for r in range(_R):
        dr = dinv0[:, r:r + 1]
        p = s_ref[:, r, :] * dr + x0 * (dr * dr)
        acc = acc + jnp.dot(p, wrel_ref[r], preferred_element_type=jnp.float32)
    x1 = jnp.where(acc >= 0, acc, 0.1 * acc)

    def head(w_ref, b_ref):
        z = jnp.dot(x1, w_ref[...], preferred_element_type=jnp.float32) + b_ref[...]
        m = jnp.max(z, axis=1, keepdims=True)
        e = jnp.exp(z - m)
        return z - m - jnp.log(jnp.sum(e, axis=1, keepdims=True))

    lg_ref[...] = head(wg_ref, bg_ref)
    ls_ref[...] = head(ws_ref, bs_ref)


def _tc_call(s, deg0, x0, W_rel, W_0, W_g, b_g, W_s, b_s):
    out_shape = [
        jax.ShapeDtypeStruct((_B, _NG), jnp.float32),
        jax.ShapeDtypeStruct((_B, _NS), jnp.float32),
    ]
    return pl.pallas_call(_tc_body, out_shape=out_shape)(
        s, deg0, x0, W_rel, W_0, W_g,
        b_g.reshape(1, _NG), W_s, b_s.reshape(1, _NS))


def kernel(x, edge_index, edge_type, W_rel, W_0, W_g, b_g, W_s, b_s):
    s_flat, deg0 = _sc_gather_sums(x, edge_index, edge_type)
    s = s_flat.reshape(_B, _R, _D)
    lg, ls = _tc_call(s, deg0, x[:, 0, :], W_rel, W_0, W_g, b_g, W_s, b_s)
    return lg, ls


# speedup vs baseline: 1.6571x; 1.0025x over previous
import jax
import jax.numpy as jnp
from jax import lax
from jax.experimental import pallas as pl
from jax.experimental.pallas import tpu as pltpu
from jax.experimental.pallas import tpu_sc as plsc

_B, _N, _E, _D, _R = 32, 512, 8192, 256, 8
_NG, _NS = 10000, 2000
_L = 16


def _sc_body(x2_hbm, ei_hbm, et_hbm, s_hbm, deg0_hbm,
             src_v, dst_v, et_v, histd_v, srcl_v, etl_v, s_v, rows_v, d0_v,
             sem):
    c = lax.axis_index("c")
    sx = lax.axis_index("s")
    w = sx * 2 + c

    pltpu.sync_copy(ei_hbm.at[w, 0], src_v)
    pltpu.sync_copy(ei_hbm.at[w, 1], dst_v)
    pltpu.sync_copy(et_hbm.at[w], et_v)

    zeros = jnp.zeros((_L,), jnp.float32)

    @plsc.parallel_loop(0, (_R * _N) // _L, 1, unroll=4)
    def _(i):
        histd_v[pl.ds(i * _L, _L)] = zeros

    @plsc.parallel_loop(0, (_R * _D) // _L, 1, unroll=4)
    def _(i):
        s_v[pl.ds(i * _L, _L)] = zeros

    ones = jnp.ones((_L,), jnp.float32)
    lanes = lax.iota(jnp.int32, _L)

    @plsc.parallel_loop(0, _E // _L, 1, unroll=4,
                        carry=jnp.zeros((_L,), jnp.int32))
    def cntv(i, cnt):
        off = i * _L
        sv = src_v[pl.ds(off, _L)]
        dv = dst_v[pl.ds(off, _L)]
        tv = et_v[pl.ds(off, _L)]
        plsc.addupdate_scatter(histd_v, [tv * _N + dv], ones)
        m = dv == 0
        pos = cnt + plsc.cumsum(m.astype(jnp.int32)) - 1
        plsc.store_scatter(srcl_v, [pos], sv, mask=m)
        plsc.store_scatter(etl_v, [pos], tv, mask=m)
        return cnt + plsc.all_reduce_population_count(m)

    nlist = jnp.max(cntv)

    def chunk_body(k, cr):
        off = k * _L
        sl = srcl_v[pl.ds(off, _L)]
        tl = etl_v[pl.ds(off, _L)]
        act = (off + lanes) < nlist
        sl = jnp.where(act, sl, 0)
        tl = jnp.where(act, tl, 0)
        d = plsc.load_gather(histd_v, [tl * _N + sl]) + 1.0
        di = plsc.bitcast(d, jnp.int32)
        y = plsc.bitcast(jnp.int32(0x5F3759DF) - (di >> 1), jnp.float32)
        half = 0.5 * d
        y = y * (1.5 - half * y * y)
        y = y * (1.5 - half * y * y)
        y = y * (1.5 - half * y * y)
        wgt = jnp.where(act, y, jnp.float32(0.0))
        gidx = jnp.where(act, w * _N + sl, 0)
        pltpu.async_copy(x2_hbm.at[gidx], rows_v, sem).wait()

        @plsc.parallel_loop(0, _D, 1, unroll=4)
        def _(cc):
            colv = plsc.load_gather(rows_v, [lanes, jnp.broadcast_to(cc, (_L,))])
            plsc.addupdate_scatter(s_v, [tl * _D + cc], wgt * colv, mask=act)

        return cr

    lax.fori_loop(0, (nlist + _L - 1) // _L, chunk_body, 0)

    d0_v[...] = plsc.load_gather(histd_v, [jnp.minimum(lanes * _N, _R * _N - 1)])
    pltpu.sync_copy(d0_v, deg0_hbm.at[w])
    pltpu.sync_copy(s_v, s_hbm.at[w])


def _sc_gather_sums(x, edge_index, edge_type):
    x2 = x.reshape(_B * _N, _D)
    mesh = plsc.VectorSubcoreMesh(core_axis_name="c", subcore_axis_name="s")
    fn = pl.kernel(
        _sc_body,
        out_type=[
            jax.ShapeDtypeStruct((_B, _R * _D), jnp.float32),
            jax.ShapeDtypeStruct((_B, _L), jnp.float32),
        ],
        mesh=mesh,
        compiler_params=pltpu.CompilerParams(needs_layout_passes=False),
        scratch_types=[
            pltpu.VMEM((_E,), jnp.int32),
            pltpu.VMEM((_E,), jnp.int32),
            pltpu.VMEM((_E,), jnp.int32),
            pltpu.VMEM((_R * _N,), jnp.float32),
            pltpu.VMEM((_E,), jnp.int32),
            pltpu.VMEM((_E,), jnp.int32),
            pltpu.VMEM((_R * _D,), jnp.float32),
            pltpu.VMEM((_L, _D), jnp.float32),
            pltpu.VMEM((_L,), jnp.float32),
            pltpu.SemaphoreType.DMA,
        ],
    )
    return fn(x2, edge_index, edge_type)


def _tc_body(s_ref, d0_ref, x0_ref, wrel_ref, w0_ref,
             wg_ref, bg_ref, ws_ref, bs_ref, lg_ref, ls_ref):
    dinv0 = lax.rsqrt(d0_ref[...] + 1.0)
    x0 = x0_ref[...]
    acc = jnp.dot(x0, w0_ref[...], preferred_element_type=jnp.float32)
    for r in range(_R):
        dr = dinv0[:, r:r + 1]
        p = s_ref[:, r, :] * dr + x0 * (dr * dr)
        acc = acc + jnp.dot(p, wrel_ref[r], preferred_element_type=jnp.float32)
    x1 = jnp.where(acc >= 0, acc, 0.1 * acc)

    def head(w_ref, b_ref):
        z = jnp.dot(x1, w_ref[...], preferred_element_type=jnp.float32) + b_ref[...]
        m = jnp.max(z, axis=1, keepdims=True)
        e = jnp.exp(z - m)
        return z - m - jnp.log(jnp.sum(e, axis=1, keepdims=True))

    lg_ref[...] = head(wg_ref, bg_ref)
    ls_ref[...] = head(ws_ref, bs_ref)


def _tc_call(s, deg0, x0, W_rel, W_0, W_g, b_g, W_s, b_s):
    out_shape = [
        jax.ShapeDtypeStruct((_B, _NG), jnp.float32),
        jax.ShapeDtypeStruct((_B, _NS), jnp.float32),
    ]
    return pl.pallas_call(_tc_body, out_shape=out_shape)(
        s, deg0, x0, W_rel, W_0, W_g,
        b_g.reshape(1, _NG), W_s, b_s.reshape(1, _NS))


def kernel(x, edge_index, edge_type, W_rel, W_0, W_g, b_g, W_s, b_s):
    s_flat, deg0 = _sc_gather_sums(x, edge_index, edge_type)
    s = s_flat.reshape(_B, _R, _D)
    lg, ls = _tc_call(s, deg0, x[:, 0, :], W_rel, W_0, W_g, b_g, W_s, b_s)
    return lg, ls
